# Initial kernel scaffold; baseline (speedup 1.0000x reference)
import jax
import jax.numpy as jnp
from jax.experimental import pallas as pl


def kernel(node_attr, edge_index, W, b):
    raise NotImplementedError("write your pallas kernel here")


# speedup vs baseline: 4.8692x; 4.8692x over previous
import functools
import jax
import jax.numpy as jnp
from jax import lax
from jax.experimental import pallas as pl
from jax.experimental.pallas import tpu as pltpu
from jax.experimental.pallas import tpu_sc as plsc

N = 10000
E = 320000
D = 128
NP = 10240
DA = 144
NC = 2
NS = 16
EPT = E // (NC * NS)
CH = 80
STEPS = EPT // CH
RPT = NP // NS


def _mm_body(na_ref, w_ref, b_ref, o_ref):
    acc = jnp.dot(na_ref[...], w_ref[...], preferred_element_type=jnp.float32)
    acc = acc + b_ref[...]
    ones = jnp.ones((acc.shape[0], DA - D), jnp.float32)
    o_ref[...] = jnp.concatenate([acc, ones], axis=1)


def _matmul_aug(na_p, W, b):
    blk = 1024
    grid = NP // blk
    return pl.pallas_call(
        _mm_body,
        grid=(grid,),
        in_specs=[
            pl.BlockSpec((blk, D), lambda i: (i, 0)),
            pl.BlockSpec((D, D), lambda i: (0, 0)),
            pl.BlockSpec((1, D), lambda i: (0, 0)),
        ],
        out_specs=pl.BlockSpec((blk, DA), lambda i: (i, 0)),
        out_shape=jax.ShapeDtypeStruct((NP, DA), jnp.float32),
    )(na_p, W, b.reshape(1, D))


def _sc_body(x_hbm, src_hbm, dst_hbm, z_hbm, adj_out,
             sidx_v, didx_v, rows_v, accum_s, sem):
    c = lax.axis_index("c")
    s = lax.axis_index("s")
    wid = c * NS + s

    @pl.when(s == 0)
    def _():
        pltpu.sync_copy(z_hbm, accum_s)

    plsc.subcore_barrier()

    base = wid * EPT

    def step(g, carry):
        off = base + g * CH
        pltpu.sync_copy(src_hbm.at[pl.ds(off, CH)], sidx_v)
        pltpu.sync_copy(dst_hbm.at[pl.ds(off, CH)], didx_v)
        pltpu.async_copy(x_hbm.at[sidx_v], rows_v, sem).wait()
        pltpu.sync_copy(rows_v, accum_s.at[didx_v], add=True)
        return carry

    lax.fori_loop(0, STEPS, step, 0)

    plsc.subcore_barrier()

    r0 = s * RPT
    pltpu.sync_copy(accum_s.at[pl.ds(r0, RPT)], adj_out.at[c, pl.ds(r0, RPT)])


def _sc_scatter(x_aug, src, dst, z):
    mesh = plsc.VectorSubcoreMesh(core_axis_name="c", subcore_axis_name="s")
    f = functools.partial(
        pl.kernel,
        out_type=jax.ShapeDtypeStruct((NC, NP, DA), jnp.float32),
        mesh=mesh,
        scratch_types=[
            pltpu.VMEM((CH,), jnp.int32),
            pltpu.VMEM((CH,), jnp.int32),
            pltpu.VMEM((CH, DA), jnp.float32),
            pltpu.VMEM_SHARED((NP, DA), jnp.float32),
            pltpu.SemaphoreType.DMA,
        ],
        compiler_params=pltpu.CompilerParams(use_tc_tiling_on_sc=False),
    )(_sc_body)
    return f(x_aug, src, dst, z)


def _comb_body(x_ref, a0_ref, a1_ref, o_ref):
    ssum = x_ref[:, :D] + a0_ref[:, :D] + a1_ref[:, :D]
    deg = a0_ref[:, D:D + 1] + a1_ref[:, D:D + 1] + 1.0
    o_ref[...] = ssum / deg


def _combine(x_aug, a0, a1):
    blk = 1024
    grid = NP // blk
    return pl.pallas_call(
        _comb_body,
        grid=(grid,),
        in_specs=[
            pl.BlockSpec((blk, DA), lambda i: (i, 0)),
            pl.BlockSpec((blk, DA), lambda i: (i, 0)),
            pl.BlockSpec((blk, DA), lambda i: (i, 0)),
        ],
        out_specs=pl.BlockSpec((blk, D), lambda i: (i, 0)),
        out_shape=jax.ShapeDtypeStruct((NP, D), jnp.float32),
    )(x_aug, a0, a1)


def kernel(node_attr, edge_index, W, b):
    na_p = jnp.pad(node_attr, ((0, NP - N), (0, 0)))
    x_aug = _matmul_aug(na_p, W, b)
    dst = edge_index[0]
    src = edge_index[1]
    z = jnp.zeros((NP, DA), jnp.float32)
    adj = _sc_scatter(x_aug, src, dst, z)
    out = _combine(x_aug, adj[0], adj[1])
    return out[:N]
